# Initial kernel scaffold; baseline (speedup 1.0000x reference)
import functools

import jax
import jax.numpy as jnp
from jax import lax
from jax.experimental import pallas as pl
from jax.experimental.pallas import tpu as pltpu
from jax.experimental.pallas import tpu_sc as plsc

D_ = 64
N_IDX_ = 4096 * 200
NW_ = 32
PER_W_ = N_IDX_ // NW_
CHUNK_ = 128
N_CHUNKS_ = PER_W_ // CHUNK_


def _gather_body(words_hbm, table_hbm, out_hbm, idx_v, rows_v, gsem):
    wid = lax.axis_index("s") * 2 + lax.axis_index("c")
    base = wid * PER_W_

    pltpu.sync_copy(words_hbm.at[pl.ds(base, PER_W_)], idx_v)

    def gather(j, buf):
        return pltpu.make_async_copy(
            table_hbm.at[idx_v.at[pl.ds(j * CHUNK_, CHUNK_)]],
            rows_v.at[buf],
            gsem,
        )

    gather(0, 0).start()

    def step(g):
        for b in range(2):
            j = g + b
            gather(j, b).wait()

            @pl.when(j + 1 < N_CHUNKS_)
            def _():
                gather(j + 1, 1 - b).start()

            pltpu.sync_copy(
                rows_v.at[b],
                out_hbm.at[pl.ds(base + j * CHUNK_, CHUNK_)],
            )

    pl.loop(0, N_CHUNKS_, step=2)(step)


@jax.jit
def kernel(words, weight):
    mesh = plsc.VectorSubcoreMesh(core_axis_name="c", subcore_axis_name="s")
    flat_words = words.reshape(-1)
    out = pl.kernel(
        _gather_body,
        mesh=mesh,
        out_type=jax.ShapeDtypeStruct((N_IDX_, D_), jnp.float32),
        scratch_types=[
            pltpu.VMEM((PER_W_,), jnp.int32),
            pltpu.VMEM((2, CHUNK_, D_), jnp.float32),
            pltpu.SemaphoreType.DMA,
        ],
    )(flat_words, weight)
    return out.reshape(words.shape[0], words.shape[1], D_)


# speedup vs baseline: 3.7809x; 3.7809x over previous
import functools

import jax
import jax.numpy as jnp
from jax import lax
from jax.experimental import pallas as pl
from jax.experimental.pallas import tpu as pltpu
from jax.experimental.pallas import tpu_sc as plsc

D_ = 64
N_IDX_ = 4096 * 200
NW_ = 32
PER_W_ = N_IDX_ // NW_
CHUNK_ = 128
N_CHUNKS_ = PER_W_ // CHUNK_


def _gather_body(words_hbm, table_hbm, out_hbm, idx_v, rows_v, gsem):
    wid = lax.axis_index("s") * 2 + lax.axis_index("c")
    base = wid * PER_W_

    pltpu.sync_copy(words_hbm.at[pl.ds(base, PER_W_)], idx_v)

    def gather(j, buf):
        return pltpu.make_async_copy(
            table_hbm.at[idx_v.at[pl.ds(j * CHUNK_, CHUNK_)]],
            rows_v.at[buf],
            gsem,
        )

    gather(0, 0).start()

    def step(g):
        for b in range(2):
            j = g + b
            gather(j, b).wait()

            @pl.when(j + 1 < N_CHUNKS_)
            def _():
                gather(j + 1, 1 - b).start()

            pltpu.sync_copy(
                rows_v.at[b],
                out_hbm.at[pl.ds(base + j * CHUNK_, CHUNK_)],
            )

    pl.loop(0, N_CHUNKS_, step=2)(step)


@jax.jit
def kernel(words, weight):
    mesh = plsc.VectorSubcoreMesh(core_axis_name="c", subcore_axis_name="s")
    flat_words = words.reshape(-1)
    out = pl.kernel(
        _gather_body,
        mesh=mesh,
        out_type=jax.ShapeDtypeStruct((N_IDX_, D_), jnp.float32),
        scratch_types=[
            pltpu.VMEM((PER_W_,), jnp.int32),
            pltpu.VMEM((2, CHUNK_, D_), jnp.float32),
            pltpu.SemaphoreType.DMA,
        ],
        compiler_params=pltpu.CompilerParams(use_tc_tiling_on_sc=False),
    )(flat_words, weight)
    return out.reshape(words.shape[0], words.shape[1], D_)


# speedup vs baseline: 4.2574x; 1.1260x over previous
import functools

import jax
import jax.numpy as jnp
from jax import lax
from jax.experimental import pallas as pl
from jax.experimental.pallas import tpu as pltpu
from jax.experimental.pallas import tpu_sc as plsc

D_ = 64
N_IDX_ = 4096 * 200
NW_ = 32
PER_W_ = N_IDX_ // NW_
CHUNK_ = 128
N_CHUNKS_ = PER_W_ // CHUNK_
NBUF_ = 4


def _gather_body(words_hbm, table_hbm, out_hbm, idx_v, rows_v, gsem):
    wid = lax.axis_index("s") * 2 + lax.axis_index("c")
    base = wid * PER_W_

    pltpu.sync_copy(words_hbm.at[pl.ds(base, PER_W_)], idx_v)

    def gather(j, buf):
        return pltpu.make_async_copy(
            table_hbm.at[idx_v.at[pl.ds(j * CHUNK_, CHUNK_)]],
            rows_v.at[buf],
            gsem,
        )

    for j in range(NBUF_ - 1):
        gather(j, j).start()

    def step(g):
        for b in range(NBUF_):
            j = g + b
            gather(j, b).wait()

            pltpu.sync_copy(
                rows_v.at[b],
                out_hbm.at[pl.ds(base + j * CHUNK_, CHUNK_)],
            )

            @pl.when(j + NBUF_ - 1 < N_CHUNKS_)
            def _():
                gather(j + NBUF_ - 1, (b + NBUF_ - 1) % NBUF_).start()

    pl.loop(0, N_CHUNKS_, step=NBUF_)(step)


@jax.jit
def kernel(words, weight):
    mesh = plsc.VectorSubcoreMesh(core_axis_name="c", subcore_axis_name="s")
    flat_words = words.reshape(-1)
    out = pl.kernel(
        _gather_body,
        mesh=mesh,
        out_type=jax.ShapeDtypeStruct((N_IDX_, D_), jnp.float32),
        scratch_types=[
            pltpu.VMEM((PER_W_,), jnp.int32),
            pltpu.VMEM((NBUF_, CHUNK_, D_), jnp.float32),
            pltpu.SemaphoreType.DMA,
        ],
        compiler_params=pltpu.CompilerParams(use_tc_tiling_on_sc=False),
    )(flat_words, weight)
    return out.reshape(words.shape[0], words.shape[1], D_)


# speedup vs baseline: 4.2710x; 1.0032x over previous
import functools

import jax
import jax.numpy as jnp
from jax import lax
from jax.experimental import pallas as pl
from jax.experimental.pallas import tpu as pltpu
from jax.experimental.pallas import tpu_sc as plsc

D_ = 64
N_IDX_ = 4096 * 200
NW_ = 32
PER_W_ = N_IDX_ // NW_
CHUNK_ = 256
N_CHUNKS_ = PER_W_ // CHUNK_
NBUF_ = 4


def _gather_body(words_hbm, table_hbm, out_hbm, idx_v, rows_v, gsem):
    wid = lax.axis_index("s") * 2 + lax.axis_index("c")
    base = wid * PER_W_

    pltpu.sync_copy(words_hbm.at[pl.ds(base, PER_W_)], idx_v)

    def gather(j, buf):
        return pltpu.make_async_copy(
            table_hbm.at[idx_v.at[pl.ds(j * CHUNK_, CHUNK_)]],
            rows_v.at[buf],
            gsem,
        )

    for j in range(NBUF_ - 1):
        gather(j, j).start()

    def step(g):
        for b in range(NBUF_):
            j = g + b
            gather(j, b).wait()

            pltpu.sync_copy(
                rows_v.at[b],
                out_hbm.at[pl.ds(base + j * CHUNK_, CHUNK_)],
            )

            @pl.when(j + NBUF_ - 1 < N_CHUNKS_)
            def _():
                gather(j + NBUF_ - 1, (b + NBUF_ - 1) % NBUF_).start()

    pl.loop(0, N_CHUNKS_, step=NBUF_)(step)


@jax.jit
def kernel(words, weight):
    mesh = plsc.VectorSubcoreMesh(core_axis_name="c", subcore_axis_name="s")
    flat_words = words.reshape(-1)
    out = pl.kernel(
        _gather_body,
        mesh=mesh,
        out_type=jax.ShapeDtypeStruct((N_IDX_, D_), jnp.float32),
        scratch_types=[
            pltpu.VMEM((PER_W_,), jnp.int32),
            pltpu.VMEM((NBUF_, CHUNK_, D_), jnp.float32),
            pltpu.SemaphoreType.DMA,
        ],
        compiler_params=pltpu.CompilerParams(use_tc_tiling_on_sc=False),
    )(flat_words, weight)
    return out.reshape(words.shape[0], words.shape[1], D_)


# speedup vs baseline: 4.2748x; 1.0009x over previous
import functools

import jax
import jax.numpy as jnp
from jax import lax
from jax.experimental import pallas as pl
from jax.experimental.pallas import tpu as pltpu
from jax.experimental.pallas import tpu_sc as plsc

D_ = 64
N_IDX_ = 4096 * 200
NW_ = 32
PER_W_ = N_IDX_ // NW_
CHUNK_ = 256
N_CHUNKS_ = PER_W_ // CHUNK_
NBUF_ = 4


def _gather_body(words_hbm, table_hbm, out_hbm, idx_v, rows_v, gsem):
    wid = lax.axis_index("s") * 2 + lax.axis_index("c")
    base = wid * PER_W_

    pltpu.sync_copy(words_hbm.at[pl.ds(base, PER_W_)], idx_v)

    def gather(j, buf):
        return pltpu.make_async_copy(
            table_hbm.at[idx_v.at[pl.ds(j * CHUNK_, CHUNK_)]],
            rows_v.at[buf],
            gsem,
        )

    for j in range(NBUF_ - 1):
        gather(j, j).start()

    def step(g):
        for b in range(NBUF_):
            j = g + b
            gather(j, b).wait()

            @pl.when(j + NBUF_ - 1 < N_CHUNKS_)
            def _():
                gather(j + NBUF_ - 1, (b + NBUF_ - 1) % NBUF_).start()

            pltpu.sync_copy(
                rows_v.at[b],
                out_hbm.at[pl.ds(base + j * CHUNK_, CHUNK_)],
            )

    pl.loop(0, N_CHUNKS_, step=NBUF_)(step)


@jax.jit
def kernel(words, weight):
    mesh = plsc.VectorSubcoreMesh(core_axis_name="c", subcore_axis_name="s")
    flat_words = words.reshape(-1)
    out = pl.kernel(
        _gather_body,
        mesh=mesh,
        out_type=jax.ShapeDtypeStruct((N_IDX_, D_), jnp.float32),
        scratch_types=[
            pltpu.VMEM((PER_W_,), jnp.int32),
            pltpu.VMEM((NBUF_, CHUNK_, D_), jnp.float32),
            pltpu.SemaphoreType.DMA,
        ],
        compiler_params=pltpu.CompilerParams(use_tc_tiling_on_sc=False),
    )(flat_words, weight)
    return out.reshape(words.shape[0], words.shape[1], D_)
